# baseline (speedup 1.0000x reference)
import jax
import jax.numpy as jnp
from jax.experimental import pallas as pl
from jax.experimental.pallas import tpu as pltpu


def _pool_kernel(tokp_ref, lenf_ref, emb_hbm, out_ref, emb_vmem, sem):
    c = pl.program_id(0)
    j = pl.program_id(1)
    nj = pl.num_programs(1)
    tb, _, D = out_ref.shape
    n_words = tokp_ref.shape[1]

    @pl.when(j == 0)
    def _():
        cp = pltpu.make_async_copy(emb_hbm, emb_vmem.at[:, 0], sem)
        cp.start()
        cp.wait()

    base = (c * nj + j) * tb

    G = 16

    def group_body(g, carry):
        b0 = base + g * G
        rows = [b0 + i for i in range(G)]
        accs = [None] * G
        for w in range(n_words):
            words = [tokp_ref[rows[i], w] for i in range(G)]
            for i in range(G):
                lo = emb_vmem[words[i] & 0x7FFF]
                accs[i] = lo if accs[i] is None else accs[i] + lo
            for i in range(G):
                hi = emb_vmem[jax.lax.shift_right_logical(words[i], 15)]
                accs[i] = accs[i] + hi
        for i in range(G):
            out_ref[g * G + i] = accs[i] / lenf_ref[rows[i]]
        return carry

    jax.lax.fori_loop(0, tb // G, group_body, 0)


def kernel(tok_batch, tok_lens, emb_table):
    B, L = tok_batch.shape
    V, D = emb_table.shape

    n_cores = 2
    tb = 128
    if B % (n_cores * tb) != 0:
        tb = B // n_cores
    tiles_per_core = B // (n_cores * tb)

    tok_i32 = tok_batch.astype(jnp.int32)
    if L % 2:
        tok_i32 = jnp.pad(tok_i32, ((0, 0), (0, 1)))
    tok_pack = tok_i32[:, 0::2] | (tok_i32[:, 1::2] << 15)
    lens_f32 = tok_lens.astype(jnp.float32)
    emb2 = emb_table.astype(jnp.float32)

    grid_spec = pltpu.PrefetchScalarGridSpec(
        num_scalar_prefetch=2,
        grid=(n_cores, tiles_per_core),
        in_specs=[pl.BlockSpec(memory_space=pl.ANY)],
        out_specs=pl.BlockSpec(
            (tb, 1, D), lambda c, j, tok, lf: (c * tiles_per_core + j, 0, 0)
        ),
        scratch_shapes=[
            pltpu.VMEM((V, 1, D), jnp.float32),
            pltpu.SemaphoreType.DMA,
        ],
    )

    out = pl.pallas_call(
        _pool_kernel,
        out_shape=jax.ShapeDtypeStruct((B, 1, D), jnp.float32),
        grid_spec=grid_spec,
        compiler_params=pltpu.CompilerParams(
            dimension_semantics=("parallel", "arbitrary"),
            vmem_limit_bytes=44 << 20,
        ),
    )(tok_pack, lens_f32, emb2)
    return out.reshape(B, D)
